# baseline (speedup 1.0000x reference)
import functools

import jax
import jax.numpy as jnp
from jax import lax
from jax.experimental import pallas as pl
from jax.experimental.pallas import tpu as pltpu
from jax.experimental.pallas import tpu_sc as plsc

B0, B1 = 4096, 200
D = 64
B = B0 * B1
NW = 32
PER_W = B // NW
R = 512
NCHUNK = PER_W // R

_mesh = plsc.VectorSubcoreMesh(core_axis_name="c", subcore_axis_name="s")


@functools.partial(
    pl.kernel,
    mesh=_mesh,
    out_type=jax.ShapeDtypeStruct((B, D), jnp.float32),
    scratch_types=[
        pltpu.VMEM((R,), jnp.int32),
        pltpu.VMEM((R, D), jnp.float32),
        pltpu.SemaphoreType.DMA,
    ],
    compiler_params=pltpu.CompilerParams(use_tc_tiling_on_sc=False),
)
def _lookup(idx_hbm, table_hbm, out_hbm, idx_v, rows_v, sem):
    wid = lax.axis_index("s") * 2 + lax.axis_index("c")
    base = wid * PER_W

    def body(g, carry):
        off = base + g * R
        pltpu.sync_copy(idx_hbm.at[pl.ds(off, R)], idx_v)
        pltpu.async_copy(table_hbm.at[idx_v], rows_v, sem).wait()
        pltpu.sync_copy(rows_v, out_hbm.at[pl.ds(off, R)])
        return carry

    lax.fori_loop(0, NCHUNK, body, 0)


def kernel(inputs, lookup_table):
    idx = inputs.reshape(-1).astype(jnp.int32)
    out = _lookup(idx, lookup_table)
    return out.reshape(B0, B1, D)


# speedup vs baseline: 1.0397x; 1.0397x over previous
import functools

import jax
import jax.numpy as jnp
from jax import lax
from jax.experimental import pallas as pl
from jax.experimental.pallas import tpu as pltpu
from jax.experimental.pallas import tpu_sc as plsc

B0, B1 = 4096, 200
D = 64
B = B0 * B1
NW = 32
PER_W = B // NW
R = 640
NCHUNK = PER_W // R
NPAIR = NCHUNK // 2

_mesh = plsc.VectorSubcoreMesh(core_axis_name="c", subcore_axis_name="s")


@functools.partial(
    pl.kernel,
    mesh=_mesh,
    out_type=jax.ShapeDtypeStruct((B, D), jnp.float32),
    scratch_types=[
        pltpu.VMEM((PER_W,), jnp.int32),
        pltpu.VMEM((2, R, D), jnp.float32),
        pltpu.SemaphoreType.DMA,
        pltpu.SemaphoreType.DMA,
        pltpu.SemaphoreType.DMA,
        pltpu.SemaphoreType.DMA,
    ],
    compiler_params=pltpu.CompilerParams(use_tc_tiling_on_sc=False),
)
def _lookup(idx_hbm, table_hbm, out_hbm, idx_all, rows_v, sg0, sg1, so0, so1):
    wid = lax.axis_index("s") * 2 + lax.axis_index("c")
    base = wid * PER_W
    pltpu.sync_copy(idx_hbm.at[pl.ds(base, PER_W)], idx_all)

    sg = (sg0, sg1)
    so = (so0, so1)

    def start_gather(g, b):
        pltpu.async_copy(
            table_hbm.at[idx_all.at[pl.ds(g * R, R)]], rows_v.at[b], sg[b]
        )

    def wait_gather(g, b):
        pltpu.make_async_copy(
            table_hbm.at[idx_all.at[pl.ds(g * R, R)]], rows_v.at[b], sg[b]
        ).wait()

    def start_write(g, b):
        pltpu.async_copy(rows_v.at[b], out_hbm.at[pl.ds(base + g * R, R)], so[b])

    def wait_write(g, b):
        pltpu.make_async_copy(
            rows_v.at[b], out_hbm.at[pl.ds(base + g * R, R)], so[b]
        ).wait()

    start_gather(0, 0)
    start_gather(1, 1)

    def body(p, carry):
        g = 2 * p
        wait_gather(g, 0)
        start_write(g, 0)
        wait_gather(g + 1, 1)
        start_write(g + 1, 1)
        wait_write(g, 0)
        start_gather(g + 2, 0)
        wait_write(g + 1, 1)
        start_gather(g + 3, 1)
        return carry

    lax.fori_loop(0, NPAIR - 1, body, 0)

    g = 2 * (NPAIR - 1)
    wait_gather(g, 0)
    start_write(g, 0)
    wait_gather(g + 1, 1)
    start_write(g + 1, 1)
    wait_write(g, 0)
    wait_write(g + 1, 1)


def kernel(inputs, lookup_table):
    idx = inputs.reshape(-1).astype(jnp.int32)
    out = _lookup(idx, lookup_table)
    return out.reshape(B0, B1, D)


# speedup vs baseline: 1.0645x; 1.0238x over previous
import functools

import jax
import jax.numpy as jnp
from jax import lax
from jax.experimental import pallas as pl
from jax.experimental.pallas import tpu as pltpu
from jax.experimental.pallas import tpu_sc as plsc

B0, B1 = 4096, 200
D = 64
NW = 32
BW = B0 // NW
TCH = 4
NCHUNK = B1 // TCH
NPAIR = NCHUNK // 2

_mesh = plsc.VectorSubcoreMesh(core_axis_name="c", subcore_axis_name="s")


@functools.partial(
    pl.kernel,
    mesh=_mesh,
    out_type=jax.ShapeDtypeStruct((B1, B0, D), jnp.float32),
    scratch_types=[
        pltpu.VMEM((B1, BW), jnp.int32),
        pltpu.VMEM((2, TCH, BW, D), jnp.float32),
        pltpu.SemaphoreType.DMA,
        pltpu.SemaphoreType.DMA,
        pltpu.SemaphoreType.DMA,
        pltpu.SemaphoreType.DMA,
    ],
    compiler_params=pltpu.CompilerParams(use_tc_tiling_on_sc=False),
)
def _lookup(idx_hbm, table_hbm, out_hbm, idx_v, rows_v, sg0, sg1, so0, so1):
    wid = lax.axis_index("s") * 2 + lax.axis_index("c")
    b0 = wid * BW
    pltpu.sync_copy(idx_hbm.at[:, pl.ds(b0, BW)], idx_v)

    sg = (sg0, sg1)
    so = (so0, so1)

    def start_gather(g, b):
        for i in range(TCH):
            pltpu.async_copy(
                table_hbm.at[idx_v.at[g * TCH + i]], rows_v.at[b, i], sg[b]
            )

    def wait_gather(g, b):
        for i in range(TCH):
            pltpu.make_async_copy(
                table_hbm.at[idx_v.at[g * TCH + i]], rows_v.at[b, i], sg[b]
            ).wait()

    def start_write(g, b):
        pltpu.async_copy(
            rows_v.at[b], out_hbm.at[pl.ds(g * TCH, TCH), pl.ds(b0, BW)], so[b]
        )

    def wait_write(g, b):
        pltpu.make_async_copy(
            rows_v.at[b], out_hbm.at[pl.ds(g * TCH, TCH), pl.ds(b0, BW)], so[b]
        ).wait()

    start_gather(0, 0)
    start_gather(1, 1)

    def body(p, carry):
        g = 2 * p
        wait_gather(g, 0)
        start_write(g, 0)
        wait_gather(g + 1, 1)
        start_write(g + 1, 1)
        wait_write(g, 0)
        start_gather(g + 2, 0)
        wait_write(g + 1, 1)
        start_gather(g + 3, 1)
        return carry

    lax.fori_loop(0, NPAIR - 1, body, 0)

    g = 2 * (NPAIR - 1)
    wait_gather(g, 0)
    start_write(g, 0)
    wait_gather(g + 1, 1)
    start_write(g + 1, 1)
    wait_write(g, 0)
    wait_write(g + 1, 1)


def kernel(inputs, lookup_table):
    idx_t = inputs.T
    out_t = _lookup(idx_t, lookup_table)
    return jnp.transpose(out_t, (1, 0, 2))
